# Initial kernel scaffold; baseline (speedup 1.0000x reference)
import jax
import jax.numpy as jnp
from jax.experimental import pallas as pl


def kernel(entity_ids, position_ids, token_type_ids, entity_table, dense_w, position_table, type_table, ln_gamma, ln_beta):
    raise NotImplementedError("write your pallas kernel here")


# speedup vs baseline: 3.2548x; 3.2548x over previous
import functools

import jax
import jax.numpy as jnp
from jax import lax
from jax.experimental import pallas as pl
from jax.experimental.pallas import tpu as pltpu
from jax.experimental.pallas import tpu_sc as plsc

_B, _L = 1024, 50
_V, _De, _H, _P = 100000, 128, 1024, 512
_N = _B * _L

_NC, _NS = 2, 16
_NW = _NC * _NS
_ROWS_PER_W = _N // _NW
_CHUNK = 800

_BT = 512
_NB = _N // _BT


def _sc_gather_body(ids_hbm, table_hbm, out_hbm, idx_v, rows_v, sem):
    wid = lax.axis_index("s") * _NC + lax.axis_index("c")
    base = wid * _ROWS_PER_W
    for c in range(_ROWS_PER_W // _CHUNK):
        off = base + c * _CHUNK
        pltpu.sync_copy(ids_hbm.at[pl.ds(off, _CHUNK)], idx_v)
        pltpu.async_copy(table_hbm.at[idx_v], rows_v, sem).wait()
        pltpu.sync_copy(rows_v, out_hbm.at[pl.ds(off, _CHUNK)])


def _sc_gather(ids_flat, table):
    mesh = plsc.VectorSubcoreMesh(core_axis_name="c", subcore_axis_name="s")
    return pl.kernel(
        _sc_gather_body,
        out_type=jax.ShapeDtypeStruct((_N, _De), jnp.float32),
        mesh=mesh,
        scratch_types=[
            pltpu.VMEM((_CHUNK,), jnp.int32),
            pltpu.VMEM((_CHUNK, _De), jnp.float32),
            pltpu.SemaphoreType.DMA,
        ],
    )(ids_flat, table)


def _tc_body(pos_ref, tt_ref, e_ref, w_ref, pt_ref, ty_ref, g_ref, b_ref, o_ref):
    proj = jnp.dot(
        e_ref[...].astype(jnp.bfloat16),
        w_ref[...].astype(jnp.bfloat16),
        preferred_element_type=jnp.float32,
    )
    pos = pos_ref[0, 0, :]
    oh = (lax.broadcasted_iota(jnp.int32, (_BT, _P), 1) == pos[:, None])
    p = jnp.dot(
        oh.astype(jnp.bfloat16),
        pt_ref[...].astype(jnp.bfloat16),
        preferred_element_type=jnp.float32,
    )
    tt = tt_ref[0, 0, :].astype(jnp.float32)[:, None]
    t = ty_ref[0:1, :] * (1.0 - tt) + ty_ref[1:2, :] * tt
    x = proj + p + t
    mu = jnp.mean(x, axis=-1, keepdims=True)
    var = jnp.mean(jnp.square(x - mu), axis=-1, keepdims=True)
    xn = (x - mu) * lax.rsqrt(var + 1e-12)
    o_ref[...] = xn * g_ref[...] + b_ref[...]


def _tc_fused(pos_blocks, tt_blocks, e_rows, dense_w, position_table, type_table, gamma, beta):
    return pl.pallas_call(
        _tc_body,
        grid=(_NB,),
        in_specs=[
            pl.BlockSpec((1, 1, _BT), lambda i: (i, 0, 0)),
            pl.BlockSpec((1, 1, _BT), lambda i: (i, 0, 0)),
            pl.BlockSpec((_BT, _De), lambda i: (i, 0)),
            pl.BlockSpec((_De, _H), lambda i: (0, 0)),
            pl.BlockSpec((_P, _H), lambda i: (0, 0)),
            pl.BlockSpec((2, _H), lambda i: (0, 0)),
            pl.BlockSpec((1, _H), lambda i: (0, 0)),
            pl.BlockSpec((1, _H), lambda i: (0, 0)),
        ],
        out_specs=pl.BlockSpec((_BT, _H), lambda i: (i, 0)),
        out_shape=jax.ShapeDtypeStruct((_N, _H), jnp.float32),
    )(pos_blocks, tt_blocks, e_rows, dense_w, position_table, type_table, gamma, beta)


def kernel(entity_ids, position_ids, token_type_ids, entity_table, dense_w,
           position_table, type_table, ln_gamma, ln_beta):
    ids_flat = entity_ids.reshape(_N)
    e_rows = _sc_gather(ids_flat, entity_table)
    pos_blocks = position_ids.reshape(_NB, 1, _BT)
    tt_blocks = token_type_ids.reshape(_NB, 1, _BT)
    out = _tc_fused(pos_blocks, tt_blocks, e_rows, dense_w, position_table,
                    type_table, ln_gamma.reshape(1, _H), ln_beta.reshape(1, _H))
    return out.reshape(_B, _L, _H)


# speedup vs baseline: 4.1207x; 1.2661x over previous
import functools

import jax
import jax.numpy as jnp
from jax import lax
from jax.experimental import pallas as pl
from jax.experimental.pallas import tpu as pltpu
from jax.experimental.pallas import tpu_sc as plsc

_B, _L = 1024, 50
_V, _De, _H, _P = 100000, 128, 1024, 512
_N = _B * _L

_NC, _NS = 2, 16
_NW = _NC * _NS
_ROWS_PER_W = _N // _NW
_CHUNK = 800

_BB = 16
_BT = _BB * _L
_NB = _B // _BB


def _sc_gather_body(ids_hbm, table_hbm, out_hbm, idx_v, rows_v, sem):
    wid = lax.axis_index("s") * _NC + lax.axis_index("c")
    base = wid * _ROWS_PER_W
    for c in range(_ROWS_PER_W // _CHUNK):
        off = base + c * _CHUNK
        pltpu.sync_copy(ids_hbm.at[pl.ds(off, _CHUNK)], idx_v)
        pltpu.async_copy(table_hbm.at[idx_v], rows_v, sem).wait()
        pltpu.sync_copy(rows_v, out_hbm.at[pl.ds(off, _CHUNK)])


def _sc_gather(ids_flat, table):
    mesh = plsc.VectorSubcoreMesh(core_axis_name="c", subcore_axis_name="s")
    return pl.kernel(
        _sc_gather_body,
        out_type=jax.ShapeDtypeStruct((_N, _De), jnp.float32),
        mesh=mesh,
        scratch_types=[
            pltpu.VMEM((_CHUNK,), jnp.int32),
            pltpu.VMEM((_CHUNK, _De), jnp.float32),
            pltpu.SemaphoreType.DMA,
        ],
    )(ids_flat, table)


def _tc_body(pos_ref, tt_ref, e_ref, w_ref, pt_ref, ty_ref, g_ref, b_ref, o_ref):
    proj = jnp.dot(
        e_ref[...].astype(jnp.bfloat16),
        w_ref[...].astype(jnp.bfloat16),
        preferred_element_type=jnp.float32,
    )
    pos = pos_ref[0, 0, :]
    oh = (lax.broadcasted_iota(jnp.int32, (_BT, _P), 1) == pos[:, None])
    p = jnp.dot(
        oh.astype(jnp.bfloat16),
        pt_ref[...].astype(jnp.bfloat16),
        preferred_element_type=jnp.float32,
    )
    tt = tt_ref[0, 0, :].astype(jnp.float32)[:, None]
    t = ty_ref[0:1, :] * (1.0 - tt) + ty_ref[1:2, :] * tt
    x = proj + p + t
    mu = jnp.mean(x, axis=-1, keepdims=True)
    var = jnp.mean(jnp.square(x - mu), axis=-1, keepdims=True)
    xn = (x - mu) * lax.rsqrt(var + 1e-12)
    o_ref[...] = (xn * g_ref[...] + b_ref[...]).reshape(_BB, _L, _H)


def _tc_fused(pos_blocks, tt_blocks, e_rows, dense_w, position_table, type_table, gamma, beta):
    return pl.pallas_call(
        _tc_body,
        grid=(_NB,),
        in_specs=[
            pl.BlockSpec((1, 1, _BT), lambda i: (i, 0, 0)),
            pl.BlockSpec((1, 1, _BT), lambda i: (i, 0, 0)),
            pl.BlockSpec((_BT, _De), lambda i: (i, 0)),
            pl.BlockSpec((_De, _H), lambda i: (0, 0)),
            pl.BlockSpec((_P, _H), lambda i: (0, 0)),
            pl.BlockSpec((2, _H), lambda i: (0, 0)),
            pl.BlockSpec((1, _H), lambda i: (0, 0)),
            pl.BlockSpec((1, _H), lambda i: (0, 0)),
        ],
        out_specs=pl.BlockSpec((_BB, _L, _H), lambda i: (i, 0, 0)),
        out_shape=jax.ShapeDtypeStruct((_B, _L, _H), jnp.float32),
    )(pos_blocks, tt_blocks, e_rows, dense_w, position_table, type_table, gamma, beta)


def kernel(entity_ids, position_ids, token_type_ids, entity_table, dense_w,
           position_table, type_table, ln_gamma, ln_beta):
    ids_flat = entity_ids.reshape(_N)
    e_rows = _sc_gather(ids_flat, entity_table)
    pos_blocks = position_ids.reshape(_NB, 1, _BT)
    tt_blocks = token_type_ids.reshape(_NB, 1, _BT)
    return _tc_fused(pos_blocks, tt_blocks, e_rows, dense_w, position_table,
                     type_table, ln_gamma.reshape(1, _H), ln_beta.reshape(1, _H))


# speedup vs baseline: 8.3369x; 2.0232x over previous
import jax
import jax.numpy as jnp
from jax import lax
from jax.experimental import pallas as pl
from jax.experimental.pallas import tpu as pltpu
from jax.experimental.pallas import tpu_sc as plsc

_B, _L = 1024, 50
_V, _De, _H, _P = 100000, 128, 1024, 512
_N = _B * _L

_NC, _NS = 2, 16
_NW = _NC * _NS
_ROWS_PER_W = _N // _NW
_CHUNK = 800


def _sc_gather_body(ids_hbm, table_hbm, out_hbm, idx_v, rows_v, sem):
    wid = lax.axis_index("s") * _NC + lax.axis_index("c")
    base = wid * _ROWS_PER_W
    for c in range(_ROWS_PER_W // _CHUNK):
        off = base + c * _CHUNK
        pltpu.sync_copy(ids_hbm.at[pl.ds(off, _CHUNK)], idx_v)
        pltpu.async_copy(table_hbm.at[idx_v], rows_v, sem).wait()
        pltpu.sync_copy(rows_v, out_hbm.at[pl.ds(off, _CHUNK)])


def _sc_gather(ids_flat, table):
    mesh = plsc.VectorSubcoreMesh(core_axis_name="c", subcore_axis_name="s")
    return pl.kernel(
        _sc_gather_body,
        out_type=jax.ShapeDtypeStruct((_N, _De), jnp.float32),
        mesh=mesh,
        scratch_types=[
            pltpu.VMEM((_CHUNK,), jnp.int32),
            pltpu.VMEM((_CHUNK, _De), jnp.float32),
            pltpu.SemaphoreType.DMA,
        ],
    )(ids_flat, table)


def _tc_body(pos_ref, tt_ref, e_ref, w_ref, pt_ref, ty_ref, g_ref, b_ref, o_ref):
    proj = jnp.dot(
        e_ref[...].astype(jnp.bfloat16),
        w_ref[...].astype(jnp.bfloat16),
        preferred_element_type=jnp.float32,
    )
    pos = pos_ref[0, 0, :]
    oh = (lax.broadcasted_iota(jnp.int32, (_B, _P), 1) == pos[:, None])
    p = jnp.dot(
        oh.astype(jnp.bfloat16),
        pt_ref[...].astype(jnp.bfloat16),
        preferred_element_type=jnp.float32,
    )
    tt = tt_ref[0, 0, :].astype(jnp.float32)[:, None]
    t = ty_ref[0:1, :] * (1.0 - tt) + ty_ref[1:2, :] * tt
    x = proj + p + t
    mu = jnp.mean(x, axis=-1, keepdims=True)
    var = jnp.mean(jnp.square(x - mu), axis=-1, keepdims=True)
    xn = (x - mu) * lax.rsqrt(var + 1e-12)
    o_ref[...] = (xn * g_ref[...] + b_ref[...]).reshape(1, _B, _H)


def _tc_fused(pos_blocks, tt_blocks, e_rows, dense_w, position_table, type_table, gamma, beta):
    return pl.pallas_call(
        _tc_body,
        grid=(_L,),
        in_specs=[
            pl.BlockSpec((1, 1, _B), lambda i: (i, 0, 0)),
            pl.BlockSpec((1, 1, _B), lambda i: (i, 0, 0)),
            pl.BlockSpec((_B, _De), lambda i: (i, 0)),
            pl.BlockSpec((_De, _H), lambda i: (0, 0)),
            pl.BlockSpec((_P, _H), lambda i: (0, 0)),
            pl.BlockSpec((2, _H), lambda i: (0, 0)),
            pl.BlockSpec((1, _H), lambda i: (0, 0)),
            pl.BlockSpec((1, _H), lambda i: (0, 0)),
        ],
        out_specs=pl.BlockSpec((1, _B, _H), lambda i: (i, 0, 0)),
        out_shape=jax.ShapeDtypeStruct((_L, _B, _H), jnp.float32),
    )(pos_blocks, tt_blocks, e_rows, dense_w, position_table, type_table, gamma, beta)


def kernel(entity_ids, position_ids, token_type_ids, entity_table, dense_w,
           position_table, type_table, ln_gamma, ln_beta):
    ids_lb = entity_ids.T.reshape(_N)
    e_rows = _sc_gather(ids_lb, entity_table)
    pos_blocks = position_ids.T.reshape(_L, 1, _B)
    tt_blocks = token_type_ids.T.reshape(_L, 1, _B)
    out_lb = _tc_fused(pos_blocks, tt_blocks, e_rows, dense_w, position_table,
                       type_table, ln_gamma.reshape(1, _H), ln_beta.reshape(1, _H))
    return jnp.transpose(out_lb, (1, 0, 2))


# speedup vs baseline: 8.8695x; 1.0639x over previous
import jax
import jax.numpy as jnp
from jax import lax
from jax.experimental import pallas as pl
from jax.experimental.pallas import tpu as pltpu
from jax.experimental.pallas import tpu_sc as plsc

_B, _L = 1024, 50
_V, _De, _H, _P = 100000, 128, 1024, 512
_N = _B * _L
_LH = _L // 2
_NH = _B * _LH

_NC, _NS = 2, 16
_NW = _NC * _NS
_ROWS_PER_W = _NH // _NW
_CHUNK = 800


def _sc_gather_body(ids_hbm, table_hbm, out_hbm, idx_v, rows_v, sem):
    wid = lax.axis_index("s") * _NC + lax.axis_index("c")
    base = wid * _ROWS_PER_W
    for c in range(_ROWS_PER_W // _CHUNK):
        off = base + c * _CHUNK
        pltpu.sync_copy(ids_hbm.at[pl.ds(off, _CHUNK)], idx_v)
        pltpu.async_copy(table_hbm.at[idx_v], rows_v, sem).wait()
        pltpu.sync_copy(rows_v, out_hbm.at[pl.ds(off, _CHUNK)])


def _sc_gather(ids_half, table):
    mesh = plsc.VectorSubcoreMesh(core_axis_name="c", subcore_axis_name="s")
    return pl.kernel(
        _sc_gather_body,
        out_type=jax.ShapeDtypeStruct((_NH, _De), jnp.float32),
        mesh=mesh,
        scratch_types=[
            pltpu.VMEM((_CHUNK,), jnp.int32),
            pltpu.VMEM((_CHUNK, _De), jnp.float32),
            pltpu.SemaphoreType.DMA,
        ],
    )(ids_half, table)


def _tc_body(pos_ref, tt_ref, e_ref, w_ref, pt_ref, ty_ref, g_ref, b_ref, o_ref):
    proj = jnp.dot(e_ref[...].astype(jnp.bfloat16), w_ref[...],
                   preferred_element_type=jnp.float32)
    pos = pos_ref[0, 0, :]
    oh = (lax.broadcasted_iota(jnp.int32, (_B, _P), 1) == pos[:, None])
    p = jnp.dot(oh.astype(jnp.bfloat16), pt_ref[...],
                preferred_element_type=jnp.float32)
    tt = tt_ref[0, 0, :].astype(jnp.float32)[:, None]
    t = ty_ref[0:1, :] + tt * (ty_ref[1:2, :] - ty_ref[0:1, :])
    x = proj + p + t
    mu = jnp.mean(x, axis=-1, keepdims=True)
    xc = x - mu
    var = jnp.mean(xc * xc, axis=-1, keepdims=True)
    xn = xc * lax.rsqrt(var + 1e-12)
    o_ref[...] = (xn * g_ref[...] + b_ref[...]).reshape(1, _B, _H)


def _tc_body_alias(prev_ref, pos_ref, tt_ref, e_ref, w_ref, pt_ref, ty_ref,
                   g_ref, b_ref, o_ref):
    del prev_ref
    _tc_body(pos_ref, tt_ref, e_ref, w_ref, pt_ref, ty_ref, g_ref, b_ref, o_ref)


_DATA_SPECS = [
    pl.BlockSpec((1, 1, _B), lambda i: (i, 0, 0)),
    pl.BlockSpec((1, 1, _B), lambda i: (i, 0, 0)),
    pl.BlockSpec((_B, _De), lambda i: (i, 0)),
    pl.BlockSpec((_De, _H), lambda i: (0, 0)),
    pl.BlockSpec((_P, _H), lambda i: (0, 0)),
    pl.BlockSpec((2, _H), lambda i: (0, 0)),
    pl.BlockSpec((1, _H), lambda i: (0, 0)),
    pl.BlockSpec((1, _H), lambda i: (0, 0)),
]


def _tc_half0(pos_blocks, tt_blocks, e_rows, *tables):
    return pl.pallas_call(
        _tc_body,
        grid=(_LH,),
        in_specs=_DATA_SPECS,
        out_specs=pl.BlockSpec((1, _B, _H), lambda i: (i, 0, 0)),
        out_shape=jax.ShapeDtypeStruct((_L, _B, _H), jnp.float32),
    )(pos_blocks, tt_blocks, e_rows, *tables)


def _tc_half1(prev, pos_blocks, tt_blocks, e_rows, *tables):
    return pl.pallas_call(
        _tc_body_alias,
        grid=(_LH,),
        in_specs=[pl.BlockSpec(memory_space=pl.ANY)] + _DATA_SPECS,
        out_specs=pl.BlockSpec((1, _B, _H), lambda i: (i + _LH, 0, 0)),
        out_shape=jax.ShapeDtypeStruct((_L, _B, _H), jnp.float32),
        input_output_aliases={0: 0},
    )(prev, pos_blocks, tt_blocks, e_rows, *tables)


def kernel(entity_ids, position_ids, token_type_ids, entity_table, dense_w,
           position_table, type_table, ln_gamma, ln_beta):
    ids_lb = entity_ids.T.reshape(_N)
    e0 = _sc_gather(ids_lb[:_NH], entity_table)
    e1 = _sc_gather(ids_lb[_NH:], entity_table)
    pos_lb = position_ids.T.reshape(_L, 1, _B)
    tt_lb = token_type_ids.T.reshape(_L, 1, _B)
    tables = (dense_w.astype(jnp.bfloat16), position_table.astype(jnp.bfloat16),
              type_table, ln_gamma.reshape(1, _H), ln_beta.reshape(1, _H))
    half0 = _tc_half0(pos_lb[:_LH], tt_lb[:_LH], e0, *tables)
    out_lb = _tc_half1(half0, pos_lb[_LH:], tt_lb[_LH:], e1, *tables)
    return jnp.transpose(out_lb, (1, 0, 2))


# speedup vs baseline: 9.1659x; 1.0334x over previous
import jax
import jax.numpy as jnp
from jax import lax
from jax.experimental import pallas as pl
from jax.experimental.pallas import tpu as pltpu
from jax.experimental.pallas import tpu_sc as plsc

_B, _L = 1024, 50
_V, _De, _H, _P = 100000, 128, 1024, 512
_N = _B * _L
_L0 = 10
_L1 = _L - _L0
_TBL = 2
_BT = _TBL * _B

_NC, _NS = 2, 16
_NW = _NC * _NS


def _sc_gather_body(n_rows, ids_hbm, table_hbm, out_hbm, idx_v, rows_v, sem):
    rows_per_w = n_rows // _NW
    n_chunks = -(-rows_per_w // 800)
    chunk = rows_per_w // n_chunks
    wid = lax.axis_index("s") * _NC + lax.axis_index("c")
    base = wid * rows_per_w
    for c in range(n_chunks):
        off = base + c * chunk
        pltpu.sync_copy(ids_hbm.at[pl.ds(off, chunk)], idx_v)
        pltpu.async_copy(table_hbm.at[idx_v], rows_v, sem).wait()
        pltpu.sync_copy(rows_v, out_hbm.at[pl.ds(off, chunk)])


def _sc_gather(ids_chunk, table, n_rows):
    mesh = plsc.VectorSubcoreMesh(core_axis_name="c", subcore_axis_name="s")
    rows_per_w = n_rows // _NW
    chunk = rows_per_w // (-(-rows_per_w // 800))
    return pl.kernel(
        lambda *a: _sc_gather_body(n_rows, *a),
        out_type=jax.ShapeDtypeStruct((n_rows, _De), jnp.float32),
        mesh=mesh,
        scratch_types=[
            pltpu.VMEM((chunk,), jnp.int32),
            pltpu.VMEM((chunk, _De), jnp.float32),
            pltpu.SemaphoreType.DMA,
        ],
    )(ids_chunk, table)


def _tc_body(pos_ref, tt_ref, e_ref, w_ref, pt_ref, ty_ref, g_ref, b_ref, o_ref):
    for j in range(_TBL):
        proj = jnp.dot(e_ref[pl.ds(j * _B, _B), :].astype(jnp.bfloat16),
                       w_ref[...], preferred_element_type=jnp.float32)
        pos = pos_ref[j, 0, :]
        oh = (lax.broadcasted_iota(jnp.int32, (_B, _P), 1) == pos[:, None])
        p = jnp.dot(oh.astype(jnp.bfloat16), pt_ref[...],
                    preferred_element_type=jnp.float32)
        tt = tt_ref[j, 0, :].astype(jnp.float32)[:, None]
        t = ty_ref[0:1, :] + tt * (ty_ref[1:2, :] - ty_ref[0:1, :])
        x = proj + p + t
        mu = jnp.mean(x, axis=-1, keepdims=True)
        xc = x - mu
        var = jnp.mean(xc * xc, axis=-1, keepdims=True)
        xn = xc * lax.rsqrt(var + 1e-12)
        o_ref[pl.ds(j, 1)] = (xn * g_ref[...] + b_ref[...]).reshape(1, _B, _H)


def _tc_body_alias(prev_ref, pos_ref, tt_ref, e_ref, w_ref, pt_ref, ty_ref,
                   g_ref, b_ref, o_ref):
    del prev_ref
    _tc_body(pos_ref, tt_ref, e_ref, w_ref, pt_ref, ty_ref, g_ref, b_ref, o_ref)


_DATA_SPECS = [
    pl.BlockSpec((_TBL, 1, _B), lambda i: (i, 0, 0)),
    pl.BlockSpec((_TBL, 1, _B), lambda i: (i, 0, 0)),
    pl.BlockSpec((_BT, _De), lambda i: (i, 0)),
    pl.BlockSpec((_De, _H), lambda i: (0, 0)),
    pl.BlockSpec((_P, _H), lambda i: (0, 0)),
    pl.BlockSpec((2, _H), lambda i: (0, 0)),
    pl.BlockSpec((1, _H), lambda i: (0, 0)),
    pl.BlockSpec((1, _H), lambda i: (0, 0)),
]


def _tc_chunk0(pos_blocks, tt_blocks, e_rows, *tables):
    return pl.pallas_call(
        _tc_body,
        grid=(_L0 // _TBL,),
        in_specs=_DATA_SPECS,
        out_specs=pl.BlockSpec((_TBL, _B, _H), lambda i: (i, 0, 0)),
        out_shape=jax.ShapeDtypeStruct((_L, _B, _H), jnp.float32),
    )(pos_blocks, tt_blocks, e_rows, *tables)


def _tc_chunk1(prev, pos_blocks, tt_blocks, e_rows, *tables):
    return pl.pallas_call(
        _tc_body_alias,
        grid=(_L1 // _TBL,),
        in_specs=[pl.BlockSpec(memory_space=pl.ANY)] + _DATA_SPECS,
        out_specs=pl.BlockSpec((_TBL, _B, _H), lambda i: (i + _L0 // _TBL, 0, 0)),
        out_shape=jax.ShapeDtypeStruct((_L, _B, _H), jnp.float32),
        input_output_aliases={0: 0},
    )(prev, pos_blocks, tt_blocks, e_rows, *tables)


def kernel(entity_ids, position_ids, token_type_ids, entity_table, dense_w,
           position_table, type_table, ln_gamma, ln_beta):
    n0 = _L0 * _B
    ids_lb = entity_ids.T.reshape(_N)
    e0 = _sc_gather(ids_lb[:n0], entity_table, n0)
    e1 = _sc_gather(ids_lb[n0:], entity_table, _N - n0)
    pos_lb = position_ids.T.reshape(_L, 1, _B)
    tt_lb = token_type_ids.T.reshape(_L, 1, _B)
    tables = (dense_w.astype(jnp.bfloat16), position_table.astype(jnp.bfloat16),
              type_table, ln_gamma.reshape(1, _H), ln_beta.reshape(1, _H))
    chunk0 = _tc_chunk0(pos_lb[:_L0], tt_lb[:_L0], e0, *tables)
    out_lb = _tc_chunk1(chunk0, pos_lb[_L0:], tt_lb[_L0:], e1, *tables)
    return jnp.transpose(out_lb, (1, 0, 2))
